# baseline (speedup 1.0000x reference)
import functools

import jax
import jax.numpy as jnp
from jax import lax
from jax.experimental import pallas as pl
from jax.experimental.pallas import tpu as pltpu
from jax.experimental.pallas import tpu_sc as plsc

NC = 2
NS = 16
NW = NC * NS

L = 16
BB = 8
CB = 128


def _sgpshift_body(B, C, T, V, x_hbm, s_hbm, out_hbm, sbuf, itab, xbuf, obuf,
                   sem_in0, sem_in1, sem_out0, sem_out1):
    NBT = B // BB
    NCT = C // CB
    PLANE = BB * CB
    TPW = T * NBT * NCT // NW

    wid = lax.axis_index("s") * NC + lax.axis_index("c")
    ctile = wid & (NCT - 1)
    btile = (wid >> 1) & (NBT - 1)
    t0 = wid >> 3
    TSTRIDE = NW // (NBT * NCT)

    lanes = lax.iota(jnp.int32, L)
    cV = jnp.full((L,), V, jnp.int32)
    cPLANE = jnp.full((L,), PLANE, jnp.int32)

    pltpu.sync_copy(s_hbm.at[pl.ds(ctile * CB * V, CB * V)], sbuf)
    for v in range(V):
        @pl.loop(0, CB // L)
        def _tr(clb, v=v):
            cl = clb * L
            clvec = jnp.full((L,), cl, jnp.int32) + lanes
            src = clvec * cV + jnp.full((L,), v, jnp.int32)
            itab[pl.ds(v * CB + cl, L)] = plsc.load_gather(sbuf, [src])

    def unit_slices(hbm, k):
        t = t0 + k * TSTRIDE
        return hbm.at[pl.ds(t * V, V), pl.ds(btile * BB, BB), pl.ds(ctile * CB, CB)]

    rowoff = [jnp.full((L,), r * CB, jnp.int32) for r in range(BB)]

    def gather_unit(xb, ob):
        for v in range(V):
            @pl.loop(0, CB // L)
            def _g(clb, v=v):
                cl = clb * L
                clvec = jnp.full((L,), cl, jnp.int32) + lanes
                iv = itab[pl.ds(v * CB + cl, L)]
                vals = [
                    plsc.load_gather(xb, [iv, jnp.full((L,), r, jnp.int32), clvec])
                    for r in range(BB)
                ]
                for r in range(BB):
                    ob[v, r, pl.ds(cl, L)] = vals[r]

    VH = V // 2

    def start_in(k, buf, sem):
        t = t0 + k * TSTRIDE
        bs = pl.ds(btile * BB, BB)
        cs = pl.ds(ctile * CB, CB)
        pltpu.async_copy(
            x_hbm.at[pl.ds(t * V, VH), bs, cs], buf.at[pl.ds(0, VH)], sem
        )
        pltpu.async_copy(
            x_hbm.at[pl.ds(t * V + VH, V - VH), bs, cs],
            buf.at[pl.ds(VH, V - VH)], sem
        )

    def start_out(k, buf, sem):
        pltpu.async_copy(buf, unit_slices(out_hbm, k), sem)

    def xb(i):
        return xbuf.at[i]

    def ob(i):
        return obuf.at[i]

    start_in(0, xb(0), sem_in0)

    @pl.loop(0, TPW, step=2)
    def _units(k):
        @pl.when(k + 1 < TPW)
        def _():
            start_in(k + 1, xb(1), sem_in1)

        pltpu.make_async_copy(unit_slices(x_hbm, k), xb(0), sem_in0).wait()

        @pl.when(k >= 2)
        def _():
            pltpu.make_async_copy(ob(0), unit_slices(out_hbm, k - 2), sem_out0).wait()

        gather_unit(xb(0), ob(0))
        start_out(k, ob(0), sem_out0)

        @pl.when(k + 1 < TPW)
        def _():
            @pl.when(k + 2 < TPW)
            def _():
                start_in(k + 2, xb(0), sem_in0)

            pltpu.make_async_copy(unit_slices(x_hbm, k + 1), xb(1), sem_in1).wait()

            @pl.when(k >= 1)
            def _():
                pltpu.make_async_copy(ob(1), unit_slices(out_hbm, k - 1), sem_out1).wait()

            gather_unit(xb(1), ob(1))
            start_out(k + 1, ob(1), sem_out1)

    if TPW >= 2:
        u = TPW - 2
        pltpu.make_async_copy(
            ob(u & 1), unit_slices(out_hbm, u), sem_out1 if (u & 1) else sem_out0
        ).wait()
    u = TPW - 1
    pltpu.make_async_copy(
        ob(u & 1), unit_slices(out_hbm, u), sem_out1 if (u & 1) else sem_out0
    ).wait()


def kernel(x, shift_indices):
    B, C, T, V = x.shape
    PLANE = BB * CB

    xt = jnp.transpose(x, (2, 3, 0, 1)).reshape(T * V, B, C)
    sflat = shift_indices.astype(jnp.int32).reshape(C * V)

    mesh = plsc.VectorSubcoreMesh(
        core_axis_name="c", subcore_axis_name="s", num_cores=NC, num_subcores=NS
    )
    run = pl.kernel(
        functools.partial(_sgpshift_body, B, C, T, V),
        out_type=jax.ShapeDtypeStruct((T * V, B, C), jnp.float32),
        mesh=mesh,
        scratch_types=[
            pltpu.VMEM((CB * V,), jnp.int32),
            pltpu.VMEM((V * CB,), jnp.int32),
            pltpu.VMEM((2, V, BB, CB), jnp.float32),
            pltpu.VMEM((2, V, BB, CB), jnp.float32),
            pltpu.SemaphoreType.DMA,
            pltpu.SemaphoreType.DMA,
            pltpu.SemaphoreType.DMA,
            pltpu.SemaphoreType.DMA,
        ],
        compiler_params=pltpu.CompilerParams(needs_layout_passes=False),
    )
    out_t = run(xt, sflat)
    return jnp.transpose(out_t.reshape(T, V, B, C), (2, 3, 0, 1))


# speedup vs baseline: 1.0064x; 1.0064x over previous
import functools

import jax
import jax.numpy as jnp
from jax import lax
from jax.experimental import pallas as pl
from jax.experimental.pallas import tpu as pltpu
from jax.experimental.pallas import tpu_sc as plsc

NC = 2
NS = 16
NW = NC * NS

L = 16
BB = 8
CB = 128


def _sgpshift_body(B, C, T, V, x_hbm, s_hbm, out_hbm, sbuf, itab, xbuf, obuf,
                   sem_in0, sem_in1, sem_out0, sem_out1, sem_in0b, sem_in1b):
    NBT = B // BB
    NCT = C // CB
    PLANE = BB * CB
    TPW = T * NBT * NCT // NW

    wid = lax.axis_index("s") * NC + lax.axis_index("c")
    ctile = wid & (NCT - 1)
    btile = (wid >> 1) & (NBT - 1)
    t0 = wid >> 3
    TSTRIDE = NW // (NBT * NCT)

    lanes = lax.iota(jnp.int32, L)
    cV = jnp.full((L,), V, jnp.int32)
    cPLANE = jnp.full((L,), PLANE, jnp.int32)

    pltpu.sync_copy(s_hbm.at[pl.ds(ctile * CB * V, CB * V)], sbuf)
    for v in range(V):
        @pl.loop(0, CB // L)
        def _tr(clb, v=v):
            cl = clb * L
            clvec = jnp.full((L,), cl, jnp.int32) + lanes
            src = clvec * cV + jnp.full((L,), v, jnp.int32)
            itab[pl.ds(v * CB + cl, L)] = plsc.load_gather(sbuf, [src])

    def unit_slices(hbm, k):
        t = t0 + k * TSTRIDE
        return hbm.at[pl.ds(t * V, V), pl.ds(btile * BB, BB), pl.ds(ctile * CB, CB)]

    rowoff = [jnp.full((L,), r * CB, jnp.int32) for r in range(BB)]

    def gather_range(xb, ob, v0, v1):
        for v in range(v0, v1):
            @pl.loop(0, CB // L)
            def _g(clb, v=v):
                cl = clb * L
                clvec = jnp.full((L,), cl, jnp.int32) + lanes
                iv = itab[pl.ds(v * CB + cl, L)]
                vals = [
                    plsc.load_gather(xb, [iv, jnp.full((L,), r, jnp.int32), clvec])
                    for r in range(BB)
                ]
                for r in range(BB):
                    ob[v, r, pl.ds(cl, L)] = vals[r]

    VH = V // 2

    def start_in(k, buf, sem, semb):
        t = t0 + k * TSTRIDE
        bs = pl.ds(btile * BB, BB)
        cs = pl.ds(ctile * CB, CB)
        pltpu.async_copy(
            x_hbm.at[pl.ds(t * V, VH), bs, cs], buf.at[pl.ds(0, VH)], sem
        )
        pltpu.async_copy(
            x_hbm.at[pl.ds(t * V + VH, V - VH), bs, cs],
            buf.at[pl.ds(VH, V - VH)], semb
        )

    def start_out(k, buf, sem):
        pltpu.async_copy(buf, unit_slices(out_hbm, k), sem)

    def xb(i):
        return xbuf.at[i]

    def ob(i):
        return obuf.at[i]

    def half_slices(hbm, k, lo, n):
        t = t0 + k * TSTRIDE
        return hbm.at[
            pl.ds(t * V + lo, n), pl.ds(btile * BB, BB), pl.ds(ctile * CB, CB)
        ]

    def do_unit(k, i, sem_in, sem_out, sem_inb):
        pltpu.make_async_copy(
            half_slices(x_hbm, k, 0, VH), xb(i).at[pl.ds(0, VH)], sem_in
        ).wait()
        gather_range(xb(i), ob(i), 0, VH)
        pltpu.async_copy(ob(i).at[pl.ds(0, VH)], half_slices(out_hbm, k, 0, VH), sem_out)
        pltpu.make_async_copy(
            half_slices(x_hbm, k, VH, V - VH), xb(i).at[pl.ds(VH, V - VH)], sem_inb
        ).wait()
        gather_range(xb(i), ob(i), VH, V)
        pltpu.async_copy(
            ob(i).at[pl.ds(VH, V - VH)], half_slices(out_hbm, k, VH, V - VH), sem_out
        )

    start_in(0, xb(0), sem_in0, sem_in0b)

    @pl.loop(0, TPW, step=2)
    def _units(k):
        @pl.when(k + 1 < TPW)
        def _():
            start_in(k + 1, xb(1), sem_in1, sem_in1b)

        @pl.when(k >= 2)
        def _():
            pltpu.make_async_copy(ob(0), unit_slices(out_hbm, k - 2), sem_out0).wait()

        do_unit(k, 0, sem_in0, sem_out0, sem_in0b)

        @pl.when(k + 1 < TPW)
        def _():
            @pl.when(k + 2 < TPW)
            def _():
                start_in(k + 2, xb(0), sem_in0, sem_in0b)

            @pl.when(k >= 1)
            def _():
                pltpu.make_async_copy(ob(1), unit_slices(out_hbm, k - 1), sem_out1).wait()

            do_unit(k + 1, 1, sem_in1, sem_out1, sem_in1b)

    if TPW >= 2:
        u = TPW - 2
        pltpu.make_async_copy(
            ob(u & 1), unit_slices(out_hbm, u), sem_out1 if (u & 1) else sem_out0
        ).wait()
    u = TPW - 1
    pltpu.make_async_copy(
        ob(u & 1), unit_slices(out_hbm, u), sem_out1 if (u & 1) else sem_out0
    ).wait()


def kernel(x, shift_indices):
    B, C, T, V = x.shape
    PLANE = BB * CB

    xt = jnp.transpose(x, (2, 3, 0, 1)).reshape(T * V, B, C)
    sflat = shift_indices.astype(jnp.int32).reshape(C * V)

    mesh = plsc.VectorSubcoreMesh(
        core_axis_name="c", subcore_axis_name="s", num_cores=NC, num_subcores=NS
    )
    run = pl.kernel(
        functools.partial(_sgpshift_body, B, C, T, V),
        out_type=jax.ShapeDtypeStruct((T * V, B, C), jnp.float32),
        mesh=mesh,
        scratch_types=[
            pltpu.VMEM((CB * V,), jnp.int32),
            pltpu.VMEM((V * CB,), jnp.int32),
            pltpu.VMEM((2, V, BB, CB), jnp.float32),
            pltpu.VMEM((2, V, BB, CB), jnp.float32),
            pltpu.SemaphoreType.DMA,
            pltpu.SemaphoreType.DMA,
            pltpu.SemaphoreType.DMA,
            pltpu.SemaphoreType.DMA,
            pltpu.SemaphoreType.DMA,
            pltpu.SemaphoreType.DMA,
        ],
        compiler_params=pltpu.CompilerParams(needs_layout_passes=False),
    )
    out_t = run(xt, sflat)
    return jnp.transpose(out_t.reshape(T, V, B, C), (2, 3, 0, 1))
